# baseline (speedup 1.0000x reference)
import functools

import jax
import jax.numpy as jnp
from jax.experimental import pallas as pl
from jax.experimental.pallas import tpu as pltpu

_E = 8
_TOPK = 2


def _expert_gemm_body(buf_ref, w1g_ref, w1u_ref, w2_ref, out_ref):
    f = pl.program_id(1)
    xb = buf_ref[0]
    g = jax.lax.dot_general(xb, w1g_ref[0], (((1,), (1,)), ((), ())),
                            preferred_element_type=jnp.float32)
    u = jax.lax.dot_general(xb, w1u_ref[0], (((1,), (1,)), ((), ())),
                            preferred_element_type=jnp.float32)
    act = (g * jax.nn.sigmoid(g) * u).astype(jnp.bfloat16)
    part = jax.lax.dot_general(act, w2_ref[0], (((1,), (1,)), ((), ())),
                               preferred_element_type=jnp.float32)

    @pl.when(f == 0)
    def _():
        out_ref[...] = part[None]

    @pl.when(f > 0)
    def _():
        out_ref[...] += part[None]


def _expert_gemms(buf16, w1b, w2b, C, K, dff):
    BF = 512 if dff % 512 == 0 else dff
    NF = dff // BF
    grid = (_E, NF)
    return pl.pallas_call(
        _expert_gemm_body,
        grid=grid,
        in_specs=[
            pl.BlockSpec((1, C, K), lambda e, f: (e, 0, 0)),
            pl.BlockSpec((1, BF, K), lambda e, f: (e, f, 0)),
            pl.BlockSpec((1, BF, K), lambda e, f, NF=NF: (e, NF + f, 0)),
            pl.BlockSpec((1, K, BF), lambda e, f: (e, 0, f)),
        ],
        out_specs=pl.BlockSpec((1, C, K), lambda e, f: (e, 0, 0)),
        out_shape=jax.ShapeDtypeStruct((_E, C, K), jnp.float32),
    )(buf16, w1b, w1b, w2b)


def kernel(x, router_logits, w1, w2):
    M, K = x.shape
    dff = w2.shape[2]
    C = (M * _TOPK // _E) * 3 // 2
    topk_logits, topk_ids = jax.lax.top_k(router_logits, _TOPK)
    topk_weights = jax.nn.softmax(topk_logits, axis=-1)
    flat_ids = topk_ids.reshape(-1)
    one_hot = jax.nn.one_hot(flat_ids, _E, dtype=jnp.int32)
    pos = jnp.take_along_axis(jnp.cumsum(one_hot, axis=0) - 1,
                              flat_ids[:, None], axis=1)[:, 0]
    keep = pos < C
    safe_pos = jnp.where(keep, pos, C - 1)
    xr = jnp.repeat(x, _TOPK, axis=0)
    vals = xr * keep[:, None].astype(x.dtype)
    buf = jnp.zeros((_E, C, K), x.dtype).at[flat_ids, safe_pos].add(vals)

    out_buf = _expert_gemms(buf.astype(jnp.bfloat16),
                            w1.astype(jnp.bfloat16),
                            w2.astype(jnp.bfloat16), C, K, dff)

    gathered = out_buf[flat_ids, safe_pos] * keep[:, None].astype(x.dtype)
    weighted = gathered.reshape(M, _TOPK, K) * topk_weights[:, :, None]
    return weighted.sum(axis=1)


# speedup vs baseline: 1.3479x; 1.3479x over previous
import functools

import jax
import jax.numpy as jnp
from jax.experimental import pallas as pl
from jax.experimental.pallas import tpu as pltpu

_E = 8
_TOPK = 2


def _expert_gemm_body(buf_ref, w1g_ref, w1u_ref, w2_ref, out_ref):
    f = pl.program_id(1)
    xb = buf_ref[0].astype(jnp.bfloat16)
    w1g = w1g_ref[0].astype(jnp.bfloat16)
    w1u = w1u_ref[0].astype(jnp.bfloat16)
    w2b = w2_ref[0].astype(jnp.bfloat16)
    g = jax.lax.dot_general(xb, w1g, (((1,), (1,)), ((), ())),
                            preferred_element_type=jnp.float32)
    u = jax.lax.dot_general(xb, w1u, (((1,), (1,)), ((), ())),
                            preferred_element_type=jnp.float32)
    act = (g * jax.nn.sigmoid(g) * u).astype(jnp.bfloat16)
    part = jax.lax.dot_general(act, w2b, (((1,), (1,)), ((), ())),
                               preferred_element_type=jnp.float32)

    @pl.when(f == 0)
    def _():
        out_ref[...] = part[None]

    @pl.when(f > 0)
    def _():
        out_ref[...] += part[None]


def _expert_gemms(buf16, w1b, w2b, C, K, dff):
    BF = 512 if dff % 512 == 0 else dff
    NF = dff // BF
    grid = (_E, NF)
    return pl.pallas_call(
        _expert_gemm_body,
        grid=grid,
        in_specs=[
            pl.BlockSpec((1, C, K), lambda e, f: (e, 0, 0)),
            pl.BlockSpec((1, BF, K), lambda e, f: (e, f, 0)),
            pl.BlockSpec((1, BF, K), lambda e, f, NF=NF: (e, NF + f, 0)),
            pl.BlockSpec((1, K, BF), lambda e, f: (e, 0, f)),
        ],
        out_specs=pl.BlockSpec((1, C, K), lambda e, f: (e, 0, 0)),
        out_shape=jax.ShapeDtypeStruct((_E, C, K), jnp.float32),
    )(buf16, w1b, w1b, w2b)


def kernel(x, router_logits, w1, w2):
    M, K = x.shape
    dff = w2.shape[2]
    C = (M * _TOPK // _E) * 3 // 2
    topk_logits, topk_ids = jax.lax.top_k(router_logits, _TOPK)
    topk_weights = jax.nn.softmax(topk_logits, axis=-1)
    flat_ids = topk_ids.reshape(-1)
    one_hot = jax.nn.one_hot(flat_ids, _E, dtype=jnp.int32)
    pos = jnp.take_along_axis(jnp.cumsum(one_hot, axis=0) - 1,
                              flat_ids[:, None], axis=1)[:, 0]
    keep = pos < C
    safe_pos = jnp.where(keep, pos, C - 1)
    xr = jnp.repeat(x, _TOPK, axis=0)
    vals = xr * keep[:, None].astype(x.dtype)
    buf = jnp.zeros((_E, C, K), x.dtype).at[flat_ids, safe_pos].add(vals)

    out_buf = _expert_gemms(buf, w1, w2, C, K, dff)

    gathered = out_buf[flat_ids, safe_pos] * keep[:, None].astype(x.dtype)
    weighted = gathered.reshape(M, _TOPK, K) * topk_weights[:, :, None]
    return weighted.sum(axis=1)


# speedup vs baseline: 2.1481x; 1.5937x over previous
import functools

import jax
import jax.numpy as jnp
from jax import lax
from jax.experimental import pallas as pl
from jax.experimental.pallas import tpu as pltpu
from jax.experimental.pallas import tpu_sc as plsc

_E = 8
_TOPK = 2
_NTILES = 32

_sc_mesh = plsc.VectorSubcoreMesh(core_axis_name="c", subcore_axis_name="s")


def _make_prepare(M, K, NTOT, TPW):
    @functools.partial(
        pl.kernel, mesh=_sc_mesh,
        out_type=jax.ShapeDtypeStruct((NTOT, K), jnp.float32),
        scratch_types=[
            pltpu.VMEM((TPW,), jnp.int32),
            pltpu.VMEM((TPW,), jnp.int32),
            pltpu.VMEM((TPW, K), jnp.float32),
            pltpu.SemaphoreType.DMA,
        ],
    )
    def _prepare(x_hbm, de_hbm, do_hbm, xs_hbm, idxe_v, idxo_v, rows_v, sem):
        wid = lax.axis_index("s") * 2 + lax.axis_index("c")
        base = wid * TPW
        pltpu.sync_copy(x_hbm.at[pl.ds(base, TPW)], rows_v)
        pltpu.sync_copy(de_hbm.at[pl.ds(base, TPW)], idxe_v)
        pltpu.sync_copy(do_hbm.at[pl.ds(base, TPW)], idxo_v)
        pltpu.async_copy(rows_v, xs_hbm.at[idxe_v], sem).wait()
        pltpu.async_copy(rows_v, xs_hbm.at[idxo_v], sem).wait()

    return _prepare


def _make_finalize(M, K, NROWS, CHT):
    @functools.partial(
        pl.kernel, mesh=_sc_mesh,
        out_type=jax.ShapeDtypeStruct((M, K), jnp.float32),
        scratch_types=[
            pltpu.VMEM((CHT,), jnp.int32),
            pltpu.VMEM((CHT,), jnp.int32),
            pltpu.VMEM((CHT, 16), jnp.float32),
            pltpu.VMEM((CHT, 16), jnp.float32),
            pltpu.VMEM((CHT, K), jnp.float32),
            pltpu.VMEM((CHT, K), jnp.float32),
            pltpu.VMEM((CHT, K), jnp.float32),
            pltpu.SemaphoreType.DMA,
        ],
    )
    def _finalize(outs_hbm, ge_hbm, go_hbm, we_hbm, wo_hbm, out_hbm,
                  ie_v, io_v, we_v, wo_v, re_v, ro_v, acc_v, sem):
        wid = lax.axis_index("s") * 2 + lax.axis_index("c")
        tok_per_tile = M // _NTILES
        for g in range(tok_per_tile // CHT):
            base = wid * tok_per_tile + g * CHT
            pltpu.sync_copy(ge_hbm.at[pl.ds(base, CHT)], ie_v)
            pltpu.sync_copy(go_hbm.at[pl.ds(base, CHT)], io_v)
            pltpu.sync_copy(we_hbm.at[pl.ds(base, CHT)], we_v)
            pltpu.sync_copy(wo_hbm.at[pl.ds(base, CHT)], wo_v)
            cpe = pltpu.async_copy(outs_hbm.at[ie_v], re_v, sem)
            cpo = pltpu.async_copy(outs_hbm.at[io_v], ro_v, sem)
            cpe.wait()
            cpo.wait()

            def body(j, _):
                wev = we_v[j, :]
                wov = wo_v[j, :]
                for c in range(K // 16):
                    sl = pl.ds(c * 16, 16)
                    acc_v[j, sl] = wev * re_v[j, sl] + wov * ro_v[j, sl]
                return 0

            lax.fori_loop(0, CHT, body, 0)
            pltpu.sync_copy(acc_v, out_hbm.at[pl.ds(base, CHT)])

    return _finalize


def _expert_gemm_body(buf_ref, w1g_ref, w1u_ref, w2_ref, out_ref):
    f = pl.program_id(1)
    xb = buf_ref[...].astype(jnp.bfloat16)
    w1g = w1g_ref[0].astype(jnp.bfloat16)
    w1u = w1u_ref[0].astype(jnp.bfloat16)
    w2b = w2_ref[0].astype(jnp.bfloat16)
    g = jax.lax.dot_general(xb, w1g, (((1,), (1,)), ((), ())),
                            preferred_element_type=jnp.float32)
    u = jax.lax.dot_general(xb, w1u, (((1,), (1,)), ((), ())),
                            preferred_element_type=jnp.float32)
    act = (g * jax.nn.sigmoid(g) * u).astype(jnp.bfloat16)
    part = jax.lax.dot_general(act, w2b, (((1,), (1,)), ((), ())),
                               preferred_element_type=jnp.float32)

    @pl.when(f == 0)
    def _():
        out_ref[...] = part

    @pl.when(f > 0)
    def _():
        out_ref[...] += part


def _expert_gemms(xs, w1, w2, C, K, dff):
    BF = 512 if dff % 512 == 0 else dff
    NF = dff // BF
    grid = (_E, NF)
    return pl.pallas_call(
        _expert_gemm_body,
        grid=grid,
        in_specs=[
            pl.BlockSpec((C, K), lambda e, f: (e, 0)),
            pl.BlockSpec((1, BF, K), lambda e, f: (e, f, 0)),
            pl.BlockSpec((1, BF, K), lambda e, f, NF=NF: (e, NF + f, 0)),
            pl.BlockSpec((1, K, BF), lambda e, f: (e, 0, f)),
        ],
        out_specs=pl.BlockSpec((C, K), lambda e, f: (e, 0)),
        out_shape=jax.ShapeDtypeStruct((_E * C, K), jnp.float32),
    )(xs, w1, w1, w2)


def kernel(x, router_logits, w1, w2):
    M, K = x.shape
    dff = w2.shape[2]
    C = (M * _TOPK // _E) * 3 // 2
    NROWS = _E * C
    NTOT = NROWS + 8

    topk_logits, topk_ids = jax.lax.top_k(router_logits, _TOPK)
    topk_weights = jax.nn.softmax(topk_logits, axis=-1)
    flat_ids = topk_ids.reshape(-1)
    one_hot = jax.nn.one_hot(flat_ids, _E, dtype=jnp.int32)
    pos = jnp.take_along_axis(jnp.cumsum(one_hot, axis=0) - 1,
                              flat_ids[:, None], axis=1)[:, 0]
    keep = pos < C
    e_rows = flat_ids * C + pos
    dest = jnp.where(keep, e_rows, NROWS).astype(jnp.int32)
    grows = jnp.where(keep, e_rows, 0).astype(jnp.int32)
    wflat = jnp.where(keep, topk_weights.reshape(-1), 0.0)
    de, do = dest[0::2], dest[1::2]
    ge, go = grows[0::2], grows[1::2]
    we = jnp.broadcast_to(wflat[0::2, None], (M, 16))
    wo = jnp.broadcast_to(wflat[1::2, None], (M, 16))

    xs = _make_prepare(M, K, NTOT, M // _NTILES)(x, de, do)

    out_s = _expert_gemms(xs, w1, w2, C, K, dff)

    out = _make_finalize(M, K, NROWS, 32)(out_s, ge, go, we, wo)
    return out
